# baseline (speedup 1.0000x reference)
import functools

import jax
import jax.numpy as jnp
from jax import lax
from jax.experimental import pallas as pl
from jax.experimental.pallas import tpu as pltpu
from jax.experimental.pallas import tpu_sc as plsc


def _row_losses(ct, cp, ap, at, chp, cht, bp, bt, pa_o, bce_o):
    d = jnp.transpose(cp[...]) - jnp.transpose(ct[...])

    def ce(logits, labels):
        m = jnp.max(logits, axis=0, keepdims=True)
        lse = jnp.log(jnp.sum(jnp.exp(logits - m), axis=0, keepdims=True)) + m
        ids = lax.broadcasted_iota(jnp.int32, logits.shape, 0)
        picked = jnp.sum(jnp.where(ids == labels, logits, 0.0),
                         axis=0, keepdims=True)
        return lse - picked

    pa_o[0:1, :] = jnp.sum(d * d, axis=0, keepdims=True) * (1.0 / 3.0)
    pa_o[1:2, :] = ce(jnp.transpose(ap[...]), at[...])
    pa_o[2:3, :] = ce(jnp.transpose(chp[...]), cht[...])
    bce_o[...] = ce(jnp.transpose(bp[...]), bt[...])


_L = 16
_NSUB = 16


def _sc_body(N, E, B,
             bce_h, agg_h, batch_h, pa_h, w_h, out_h,
             bsum, bcnt, seg_r, seg_a, seg_c, seg_b, seg_n, tot,
             zbuf, ones, idx16, vval, bidx, rv, av, cv, msum, mcnt, mvf,
             s_r, s_a, s_c, s_b, s_n, wv, tvv, zidx, ssem):
    cid = lax.axis_index("c")
    sid = lax.axis_index("s")
    n_per = N // _NSUB
    e_rows = (E // 128) // _NSUB
    n_rows = n_per // 128

    @pl.when(cid == 0)
    def _():
        for k in range(n_per // _L):
            zbuf[pl.ds(k * _L, _L)] = jnp.zeros((_L,), jnp.float32)
        for k in range(128 // _L):
            ones[pl.ds(k * _L, _L)] = jnp.ones((_L,), jnp.float32)
        pltpu.sync_copy(zbuf, bsum.at[pl.ds(sid * n_per, n_per)])
        pltpu.sync_copy(zbuf, bcnt.at[pl.ds(sid * n_per, n_per)])
        pltpu.sync_copy(agg_h.at[pl.ds(sid * e_rows, e_rows)], idx16)
        pltpu.sync_copy(bce_h.at[pl.ds(sid * e_rows, e_rows)], vval)

        @pl.when(sid == 0)
        def _():
            for ref in (seg_r, seg_a, seg_c, seg_b, seg_n):
                pltpu.sync_copy(zbuf.at[pl.ds(0, B)], ref)
            pltpu.sync_copy(zbuf.at[pl.ds(0, _L)], tot)

    plsc.subcore_barrier()

    @pl.when(cid == 0)
    def _():
        hs = []
        for r in range(e_rows):
            hs.append(pltpu.async_copy(vval.at[r], bsum.at[idx16.at[r]],
                                       ssem, add=True))
            hs.append(pltpu.async_copy(ones, bcnt.at[idx16.at[r]],
                                       ssem, add=True))
        pltpu.sync_copy(batch_h.at[pl.ds(sid * n_rows, n_rows)], bidx)
        pltpu.sync_copy(pa_h.at[0, pl.ds(sid * n_rows, n_rows)], rv)
        pltpu.sync_copy(pa_h.at[1, pl.ds(sid * n_rows, n_rows)], av)
        pltpu.sync_copy(pa_h.at[2, pl.ds(sid * n_rows, n_rows)], cv)
        for h in hs:
            h.wait()

    plsc.subcore_barrier()

    @pl.when(cid == 0)
    def _():
        pltpu.sync_copy(bsum.at[pl.ds(sid * n_per, n_per)], msum)
        pltpu.sync_copy(bcnt.at[pl.ds(sid * n_per, n_per)], mcnt)

        def mean_group(k, _):
            s = msum[pl.ds(k * _L, _L)]
            c = mcnt[pl.ds(k * _L, _L)]
            mvf[pl.ds(k * _L, _L)] = jnp.where(
                c > 0, 0.5 * s / jnp.maximum(c, 1.0), 0.0)
            return 0

        lax.fori_loop(0, n_per // _L, mean_group, 0)

        hs = []
        for r in range(n_rows):
            row = bidx.at[r]
            sl = pl.ds(r * 128, 128)
            hs.append(pltpu.async_copy(mvf.at[sl], seg_b.at[row], ssem,
                                       add=True))
            hs.append(pltpu.async_copy(rv.at[r], seg_r.at[row], ssem,
                                       add=True))
            hs.append(pltpu.async_copy(av.at[r], seg_a.at[row], ssem,
                                       add=True))
            hs.append(pltpu.async_copy(cv.at[r], seg_c.at[row], ssem,
                                       add=True))
            hs.append(pltpu.async_copy(ones, seg_n.at[row], ssem, add=True))
        for h in hs:
            h.wait()

    plsc.subcore_barrier()

    @pl.when((cid == 0) & (sid == 0))
    def _():
        pltpu.sync_copy(seg_r, s_r)
        pltpu.sync_copy(seg_a, s_a)
        pltpu.sync_copy(seg_c, s_c)
        pltpu.sync_copy(seg_b, s_b)
        pltpu.sync_copy(seg_n, s_n)
        pltpu.sync_copy(w_h, wv)
        acc_r = jnp.zeros((_L,), jnp.float32)
        acc_a = jnp.zeros((_L,), jnp.float32)
        acc_c = jnp.zeros((_L,), jnp.float32)
        acc_b = jnp.zeros((_L,), jnp.float32)
        for k in range(B // _L):
            sl = pl.ds(k * _L, _L)
            cn = s_n[sl]
            w = wv[sl]
            good = cn > 0
            cd = jnp.maximum(cn, 1.0)
            acc_r += jnp.where(good, s_r[sl] / cd, 0.0) * w
            acc_a += jnp.where(good, s_a[sl] / cd, 0.0) * w
            acc_c += jnp.where(good, s_c[sl] / cd, 0.0) * w
            acc_b += jnp.where(good, s_b[sl] / cd, 0.0) * w
        for x, acc in enumerate((acc_r, acc_a, acc_c, acc_b)):
            tvv[x, pl.ds(0, _L)] = acc
            zidx[x, pl.ds(0, _L)] = jnp.full((_L,), x, jnp.int32)
        for x in range(4):
            pltpu.sync_copy(tvv.at[x], tot.at[zidx.at[x]], add=True)
        pltpu.sync_copy(tot, out_h)


def _make_sc_kernel(N, E, B):
    mesh = plsc.VectorSubcoreMesh(core_axis_name="c", subcore_axis_name="s")
    n_per = N // _NSUB
    e_rows = (E // 128) // _NSUB
    n_rows = n_per // 128
    f32 = jnp.float32
    i32 = jnp.int32
    return pl.kernel(
        functools.partial(_sc_body, N, E, B),
        out_type=jax.ShapeDtypeStruct((_L,), f32),
        mesh=mesh,
        scratch_types=[
            pltpu.VMEM_SHARED((N,), f32),
            pltpu.VMEM_SHARED((N,), f32),
            pltpu.VMEM_SHARED((B,), f32),
            pltpu.VMEM_SHARED((B,), f32),
            pltpu.VMEM_SHARED((B,), f32),
            pltpu.VMEM_SHARED((B,), f32),
            pltpu.VMEM_SHARED((B,), f32),
            pltpu.VMEM_SHARED((_L,), f32),
            pltpu.VMEM((n_per,), f32),
            pltpu.VMEM((128,), f32),
            pltpu.VMEM((e_rows, 128), i32),
            pltpu.VMEM((e_rows, 128), f32),
            pltpu.VMEM((n_rows, 128), i32),
            pltpu.VMEM((n_rows, 128), f32),
            pltpu.VMEM((n_rows, 128), f32),
            pltpu.VMEM((n_rows, 128), f32),
            pltpu.VMEM((n_per,), f32),
            pltpu.VMEM((n_per,), f32),
            pltpu.VMEM((n_per,), f32),
            pltpu.VMEM((B,), f32),
            pltpu.VMEM((B,), f32),
            pltpu.VMEM((B,), f32),
            pltpu.VMEM((B,), f32),
            pltpu.VMEM((B,), f32),
            pltpu.VMEM((B,), f32),
            pltpu.VMEM((4, _L), f32),
            pltpu.VMEM((4, _L), i32),
            pltpu.SemaphoreType.DMA,
        ],
    )


def kernel(coords_true, coords_pred, atoms_pred, atoms_true,
           charges_pred, charges_true, bonds_pred, bonds_true,
           batch, bond_aggregation_index, weights):
    N = coords_true.shape[0]
    E = bonds_pred.shape[0]
    B = weights.shape[0]

    pa, bce = pl.pallas_call(
        _row_losses,
        out_shape=[
            jax.ShapeDtypeStruct((3, N), jnp.float32),
            jax.ShapeDtypeStruct((1, E), jnp.float32),
        ],
    )(
        coords_true, coords_pred,
        atoms_pred, atoms_true.astype(jnp.int32).reshape(1, N),
        charges_pred, charges_true.astype(jnp.int32).reshape(1, N),
        bonds_pred, bonds_true.astype(jnp.int32).reshape(1, E),
    )

    out = _make_sc_kernel(N, E, B)(
        bce.reshape(E // 128, 128),
        bond_aggregation_index.astype(jnp.int32).reshape(E // 128, 128),
        batch.astype(jnp.int32).reshape(N // 128, 128),
        pa.reshape(3, N // 128, 128),
        weights,
    )
    return (out[0], out[1], out[2], out[3])


# speedup vs baseline: 2.4373x; 2.4373x over previous
import functools

import jax
import jax.numpy as jnp
from jax import lax
from jax.experimental import pallas as pl
from jax.experimental.pallas import tpu as pltpu
from jax.experimental.pallas import tpu_sc as plsc


def _row_losses(ct, cp, ap, at, chp, cht, bp, bt, pa_o, bce_o):
    d = cp[...] - ct[...]

    def ce(logits, labels):
        m = jnp.max(logits, axis=0, keepdims=True)
        lse = jnp.log(jnp.sum(jnp.exp(logits - m), axis=0, keepdims=True)) + m
        ids = lax.broadcasted_iota(jnp.int32, logits.shape, 0)
        picked = jnp.sum(jnp.where(ids == labels, logits, 0.0),
                         axis=0, keepdims=True)
        return lse - picked

    pa_o[0:1, :] = jnp.sum(d * d, axis=0, keepdims=True) * (1.0 / 3.0)
    pa_o[1:2, :] = ce(ap[...], at[...])
    pa_o[2:3, :] = ce(chp[...], cht[...])
    bce_o[...] = ce(bp[...], bt[...])


_L = 16
_NSUB = 16


def _sc_body(N, E, B,
             bce_h, agg_h, batch_h, pa_h, w_h, out_h,
             bsum, bcnt, seg_r, seg_a, seg_c, seg_b, seg_n, tot,
             zbuf, ones, idx16, vval, bidx, rv, av, cv, msum, mcnt, mvf,
             s_r, s_a, s_c, s_b, s_n, wv, tvv, zidx, ssem):
    cid = lax.axis_index("c")
    sid = lax.axis_index("s")
    n_per = N // _NSUB
    e_rows = (E // 128) // _NSUB
    n_rows = n_per // 128

    @pl.when(cid == 0)
    def _():
        for k in range(n_per // _L):
            zbuf[pl.ds(k * _L, _L)] = jnp.zeros((_L,), jnp.float32)
        for k in range(128 // _L):
            ones[pl.ds(k * _L, _L)] = jnp.ones((_L,), jnp.float32)
        pltpu.sync_copy(zbuf, bsum.at[pl.ds(sid * n_per, n_per)])
        pltpu.sync_copy(zbuf, bcnt.at[pl.ds(sid * n_per, n_per)])
        pltpu.sync_copy(agg_h.at[pl.ds(sid * e_rows, e_rows)], idx16)
        pltpu.sync_copy(bce_h.at[pl.ds(sid * e_rows, e_rows)], vval)

        for i, ref in enumerate((seg_r, seg_a, seg_c, seg_b, seg_n)):
            @pl.when(sid == i)
            def _(ref=ref):
                pltpu.sync_copy(zbuf.at[pl.ds(0, B)], ref)

        @pl.when(sid == 5)
        def _():
            pltpu.sync_copy(zbuf.at[pl.ds(0, _L)], tot)

    plsc.subcore_barrier()

    @pl.when(cid == 0)
    def _():
        hs = []
        for r in range(e_rows):
            hs.append(pltpu.async_copy(vval.at[r], bsum.at[idx16.at[r]],
                                       ssem, add=True))
            hs.append(pltpu.async_copy(ones, bcnt.at[idx16.at[r]],
                                       ssem, add=True))
        pltpu.sync_copy(batch_h.at[pl.ds(sid * n_rows, n_rows)], bidx)
        pltpu.sync_copy(pa_h.at[0, pl.ds(sid * n_rows, n_rows)], rv)
        pltpu.sync_copy(pa_h.at[1, pl.ds(sid * n_rows, n_rows)], av)
        pltpu.sync_copy(pa_h.at[2, pl.ds(sid * n_rows, n_rows)], cv)
        for h in hs:
            h.wait()

    plsc.subcore_barrier()

    @pl.when(cid == 0)
    def _():
        pltpu.sync_copy(bsum.at[pl.ds(sid * n_per, n_per)], msum)
        pltpu.sync_copy(bcnt.at[pl.ds(sid * n_per, n_per)], mcnt)

        def mean_group(k, _):
            s = msum[pl.ds(k * _L, _L)]
            c = mcnt[pl.ds(k * _L, _L)]
            mvf[pl.ds(k * _L, _L)] = jnp.where(
                c > 0, 0.5 * s / jnp.maximum(c, 1.0), 0.0)
            return 0

        lax.fori_loop(0, n_per // _L, mean_group, 0)

        hs = []
        for r in range(n_rows):
            row = bidx.at[r]
            sl = pl.ds(r * 128, 128)
            hs.append(pltpu.async_copy(mvf.at[sl], seg_b.at[row], ssem,
                                       add=True))
            hs.append(pltpu.async_copy(rv.at[r], seg_r.at[row], ssem,
                                       add=True))
            hs.append(pltpu.async_copy(av.at[r], seg_a.at[row], ssem,
                                       add=True))
            hs.append(pltpu.async_copy(cv.at[r], seg_c.at[row], ssem,
                                       add=True))
            hs.append(pltpu.async_copy(ones, seg_n.at[row], ssem, add=True))
        for h in hs:
            h.wait()

    plsc.subcore_barrier()

    @pl.when(cid == 0)
    def _():
        mb = B // _NSUB
        pltpu.sync_copy(seg_r.at[pl.ds(sid * mb, mb)], s_r)
        pltpu.sync_copy(seg_a.at[pl.ds(sid * mb, mb)], s_a)
        pltpu.sync_copy(seg_c.at[pl.ds(sid * mb, mb)], s_c)
        pltpu.sync_copy(seg_b.at[pl.ds(sid * mb, mb)], s_b)
        pltpu.sync_copy(seg_n.at[pl.ds(sid * mb, mb)], s_n)
        pltpu.sync_copy(w_h.at[pl.ds(sid * mb, mb)], wv)
        sl = pl.ds(0, _L)
        cn = s_n[sl]
        w = wv[sl]
        good = cn > 0
        cd = jnp.maximum(cn, 1.0)
        accs = (jnp.where(good, s_r[sl] / cd, 0.0) * w,
                jnp.where(good, s_a[sl] / cd, 0.0) * w,
                jnp.where(good, s_c[sl] / cd, 0.0) * w,
                jnp.where(good, s_b[sl] / cd, 0.0) * w)
        for x, acc in enumerate(accs):
            tvv[x, pl.ds(0, _L)] = acc
            zidx[x, pl.ds(0, _L)] = jnp.full((_L,), x, jnp.int32)
        hs = []
        for x in range(4):
            hs.append(pltpu.async_copy(tvv.at[x], tot.at[zidx.at[x]], ssem,
                                       add=True))
        for h in hs:
            h.wait()

    plsc.subcore_barrier()

    @pl.when((cid == 0) & (sid == 0))
    def _():
        pltpu.sync_copy(tot, out_h)


def _make_sc_kernel(N, E, B):
    mesh = plsc.VectorSubcoreMesh(core_axis_name="c", subcore_axis_name="s")
    n_per = N // _NSUB
    e_rows = (E // 128) // _NSUB
    n_rows = n_per // 128
    f32 = jnp.float32
    i32 = jnp.int32
    return pl.kernel(
        functools.partial(_sc_body, N, E, B),
        out_type=jax.ShapeDtypeStruct((_L,), f32),
        mesh=mesh,
        scratch_types=[
            pltpu.VMEM_SHARED((N,), f32),
            pltpu.VMEM_SHARED((N,), f32),
            pltpu.VMEM_SHARED((B,), f32),
            pltpu.VMEM_SHARED((B,), f32),
            pltpu.VMEM_SHARED((B,), f32),
            pltpu.VMEM_SHARED((B,), f32),
            pltpu.VMEM_SHARED((B,), f32),
            pltpu.VMEM_SHARED((_L,), f32),
            pltpu.VMEM((n_per,), f32),
            pltpu.VMEM((128,), f32),
            pltpu.VMEM((e_rows, 128), i32),
            pltpu.VMEM((e_rows, 128), f32),
            pltpu.VMEM((n_rows, 128), i32),
            pltpu.VMEM((n_rows, 128), f32),
            pltpu.VMEM((n_rows, 128), f32),
            pltpu.VMEM((n_rows, 128), f32),
            pltpu.VMEM((n_per,), f32),
            pltpu.VMEM((n_per,), f32),
            pltpu.VMEM((n_per,), f32),
            pltpu.VMEM((B // _NSUB,), f32),
            pltpu.VMEM((B // _NSUB,), f32),
            pltpu.VMEM((B // _NSUB,), f32),
            pltpu.VMEM((B // _NSUB,), f32),
            pltpu.VMEM((B // _NSUB,), f32),
            pltpu.VMEM((B // _NSUB,), f32),
            pltpu.VMEM((4, _L), f32),
            pltpu.VMEM((4, _L), i32),
            pltpu.SemaphoreType.DMA,
        ],
    )


def kernel(coords_true, coords_pred, atoms_pred, atoms_true,
           charges_pred, charges_true, bonds_pred, bonds_true,
           batch, bond_aggregation_index, weights):
    N = coords_true.shape[0]
    E = bonds_pred.shape[0]
    B = weights.shape[0]

    pa, bce = pl.pallas_call(
        _row_losses,
        out_shape=[
            jax.ShapeDtypeStruct((3, N), jnp.float32),
            jax.ShapeDtypeStruct((1, E), jnp.float32),
        ],
    )(
        coords_true.T, coords_pred.T,
        atoms_pred.T, atoms_true.astype(jnp.int32).reshape(1, N),
        charges_pred.T, charges_true.astype(jnp.int32).reshape(1, N),
        bonds_pred.T, bonds_true.astype(jnp.int32).reshape(1, E),
    )

    out = _make_sc_kernel(N, E, B)(
        bce.reshape(E // 128, 128),
        bond_aggregation_index.astype(jnp.int32).reshape(E // 128, 128),
        batch.astype(jnp.int32).reshape(N // 128, 128),
        pa.reshape(3, N // 128, 128),
        weights,
    )
    return (out[0], out[1], out[2], out[3])
